# baseline (speedup 1.0000x reference)
import functools

import jax
import jax.numpy as jnp
import numpy as np
from jax import lax
from jax.experimental import pallas as pl
from jax.experimental.pallas import tpu as pltpu
from jax.experimental.pallas import tpu_sc as plsc

B = 2
N_IN = 12
KEEP = 3
H = 32

_SCORES = np.array(
    [
        [1056585764, 1059981104, 1058915320, 1057988288, 1055308516,
         1058405198, 1033450928, 1061580580, 1060302590, 1062310394,
         1051941684, 1063219490],
        [1064109712, 1063006598, 1056211448, 1062307846, 1060510548,
         1058419146, 1033307040, 1061622336, 1053762360, 1039398624,
         1020728832, 1059299640],
    ],
    dtype=np.uint32,
).view(np.float32)
_SCORES_PADDED = np.full((8, 128), 2.0, np.float32)
_SCORES_PADDED[:B, :N_IN] = _SCORES


def _select_slots(read_score):
    src = [[jnp.int32(0)] * KEEP for _ in range(B)]
    for b in range(B):
        s = [read_score(b, i) for i in range(N_IN)]
        for j in range(N_IN):
            rank = jnp.int32(0)
            for k in range(N_IN):
                before = (s[k] < s[j]) | ((s[k] == s[j]) & (k < j))
                rank = rank + jnp.where(before, 1, 0)
            for p in range(KEEP):
                sel = rank == (KEEP + p)
                src[b][p] = jnp.where(sel, jnp.int32(j), src[b][p])
    return src


def _sc_body(in_hbm, scores_hbm, out_a, scores_v, buf, sem_in, sem_out):
    wid = lax.axis_index("s") * 2 + lax.axis_index("c")

    pltpu.sync_copy(scores_hbm, scores_v)
    s_vecs = [scores_v[b, pl.ds(0, 16)] for b in range(B)]
    src = _select_slots(lambda b, i: s_vecs[b][i])

    gathers = [
        pltpu.async_copy(
            in_hbm.at[b, src[b][p], wid], buf.at[b * KEEP + p], sem_in
        )
        for b in range(B)
        for p in range(KEEP)
    ]
    for g in gathers:
        g.wait()
    stores = [
        pltpu.async_copy(buf.at[b * KEEP + p], out_a.at[b, p, wid], sem_out)
        for b in range(B)
        for p in range(KEEP)
    ]
    for s_ in stores:
        s_.wait()


def _tc_body(scores_smem, in_any, out_b, sem):
    src = _select_slots(lambda b, i: scores_smem[b, i])
    copies = [
        pltpu.make_async_copy(in_any.at[b, src[b][p]], out_b.at[b * KEEP + p], sem)
        for b in range(B)
        for p in range(KEEP)
    ]
    for c in copies:
        c.start()
    for c in copies:
        c.wait()


@jax.jit
def _gather_both(xt, scores_tile, scores_small):
    mesh = plsc.VectorSubcoreMesh(core_axis_name="c", subcore_axis_name="s")
    sc_call = pl.kernel(
        _sc_body,
        out_type=jax.ShapeDtypeStruct((B, KEEP, H, 32, 384), jnp.float32),
        mesh=mesh,
        scratch_types=[
            pltpu.VMEM((8, 128), jnp.float32),
            pltpu.VMEM((B * KEEP, 32, 384), jnp.float32),
            pltpu.SemaphoreType.DMA,
            pltpu.SemaphoreType.DMA,
        ],
        compiler_params=pltpu.CompilerParams(use_tc_tiling_on_sc=True),
    )
    ya = sc_call(xt, scores_tile)
    yb = pl.pallas_call(
        _tc_body,
        out_shape=jax.ShapeDtypeStruct((B * KEEP, H, 32, 384), jnp.float32),
        in_specs=[
            pl.BlockSpec(memory_space=pltpu.SMEM),
            pl.BlockSpec(memory_space=pl.ANY),
        ],
        out_specs=pl.BlockSpec(memory_space=pl.ANY),
        scratch_shapes=[pltpu.SemaphoreType.DMA],
    )(scores_small, xt)
    return ya, yb


def kernel(image_latent):
    xt = jnp.transpose(image_latent, (0, 1, 3, 4, 2))
    ya, yb = _gather_both(
        xt, jnp.asarray(_SCORES_PADDED), jnp.asarray(_SCORES)
    )
    return (
        jnp.transpose(ya, (0, 1, 4, 2, 3)),
        jnp.transpose(yb, (0, 3, 1, 2)),
    )


# speedup vs baseline: 8.7902x; 8.7902x over previous
import functools

import jax
import jax.numpy as jnp
import numpy as np
from jax import lax
from jax.experimental import pallas as pl
from jax.experimental.pallas import tpu as pltpu
from jax.experimental.pallas import tpu_sc as plsc

B = 2
N_IN = 12
KEEP = 3
H = 32

_SCORES = np.array(
    [
        [1056585764, 1059981104, 1058915320, 1057988288, 1055308516,
         1058405198, 1033450928, 1061580580, 1060302590, 1062310394,
         1051941684, 1063219490],
        [1064109712, 1063006598, 1056211448, 1062307846, 1060510548,
         1058419146, 1033307040, 1061622336, 1053762360, 1039398624,
         1020728832, 1059299640],
    ],
    dtype=np.uint32,
).view(np.float32)
_SCORES_PADDED = np.full((8, 128), 2.0, np.float32)
_SCORES_PADDED[:B, :N_IN] = _SCORES


def _select_slots(read_score):
    src = [[jnp.int32(0)] * KEEP for _ in range(B)]
    for b in range(B):
        s = [read_score(b, i) for i in range(N_IN)]
        for j in range(N_IN):
            rank = jnp.int32(0)
            for k in range(N_IN):
                before = (s[k] < s[j]) | ((s[k] == s[j]) & (k < j))
                rank = rank + jnp.where(before, 1, 0)
            for p in range(KEEP):
                sel = rank == (KEEP + p)
                src[b][p] = jnp.where(sel, jnp.int32(j), src[b][p])
    return src


def _sc_body(in_hbm, scores_hbm, out_a, scores_v, buf, sem_in, sem_out):
    wid = lax.axis_index("s") * 2 + lax.axis_index("c")

    pltpu.sync_copy(scores_hbm, scores_v)
    s_vecs = [scores_v[b, pl.ds(0, 16)] for b in range(B)]
    src = _select_slots(lambda b, i: s_vecs[b][i])

    gathers = [
        pltpu.async_copy(
            in_hbm.at[b, src[b][p], wid], buf.at[b * KEEP + p], sem_in
        )
        for b in range(B)
        for p in range(KEEP)
    ]
    for g in gathers:
        g.wait()
    stores = [
        pltpu.async_copy(buf.at[b * KEEP + p], out_a.at[b, p, wid], sem_out)
        for b in range(B)
        for p in range(KEEP)
    ]
    for s_ in stores:
        s_.wait()


def _tc_body(scores_smem, in_any, out_b, buf, gsem, ssem):
    src = _select_slots(lambda b, i: scores_smem[b, i])
    jobs = [(b, p) for b in range(B) for p in range(KEEP)]
    n = len(jobs)

    def mk_gather(j):
        b, p = jobs[j]
        return pltpu.make_async_copy(
            in_any.at[b, src[b][p]], buf.at[j % 2], gsem.at[j % 2]
        )

    def mk_store(j):
        b, p = jobs[j]
        return pltpu.make_async_copy(
            buf.at[j % 2], out_b.at[b * KEEP + p], ssem.at[j % 2]
        )

    stores = [None] * n
    g = mk_gather(0)
    g.start()
    for j in range(n):
        nxt = None
        if j + 1 < n:
            if j - 1 >= 0:
                stores[j - 1].wait()
            nxt = mk_gather(j + 1)
            nxt.start()
        g.wait()
        stores[j] = mk_store(j)
        stores[j].start()
        g = nxt
    stores[n - 2].wait()
    stores[n - 1].wait()


@jax.jit
def _gather_both(xt, scores_tile, scores_small):
    mesh = plsc.VectorSubcoreMesh(core_axis_name="c", subcore_axis_name="s")
    sc_call = pl.kernel(
        _sc_body,
        out_type=jax.ShapeDtypeStruct((B, KEEP, H, 32, 384), jnp.float32),
        mesh=mesh,
        scratch_types=[
            pltpu.VMEM((8, 128), jnp.float32),
            pltpu.VMEM((B * KEEP, 32, 384), jnp.float32),
            pltpu.SemaphoreType.DMA,
            pltpu.SemaphoreType.DMA,
        ],
        compiler_params=pltpu.CompilerParams(use_tc_tiling_on_sc=True),
    )
    ya = sc_call(xt, scores_tile)
    yb = pl.pallas_call(
        _tc_body,
        out_shape=jax.ShapeDtypeStruct((B * KEEP, H, 32, 384), jnp.float32),
        in_specs=[
            pl.BlockSpec(memory_space=pltpu.SMEM),
            pl.BlockSpec(memory_space=pl.ANY),
        ],
        out_specs=pl.BlockSpec(memory_space=pl.ANY),
        scratch_shapes=[
            pltpu.VMEM((2, H, 32, 384), jnp.float32),
            pltpu.SemaphoreType.DMA((2,)),
            pltpu.SemaphoreType.DMA((2,)),
        ],
    )(scores_small, xt)
    return ya, yb


def kernel(image_latent):
    xt = jnp.transpose(image_latent, (0, 1, 3, 4, 2))
    ya, yb = _gather_both(
        xt, jnp.asarray(_SCORES_PADDED), jnp.asarray(_SCORES)
    )
    return (
        jnp.transpose(ya, (0, 1, 4, 2, 3)),
        jnp.transpose(yb, (0, 3, 1, 2)),
    )


# speedup vs baseline: 9.3910x; 1.0683x over previous
import functools

import jax
import jax.numpy as jnp
import numpy as np
from jax import lax
from jax.experimental import pallas as pl
from jax.experimental.pallas import tpu as pltpu
from jax.experimental.pallas import tpu_sc as plsc

B = 2
N_IN = 12
KEEP = 3
H = 32
HC = 2
N_CHUNKS = H // HC
JOBS_PER_SUBCORE = B * KEEP * N_CHUNKS // 32

_SCORES = np.array(
    [
        [1056585764, 1059981104, 1058915320, 1057988288, 1055308516,
         1058405198, 1033450928, 1061580580, 1060302590, 1062310394,
         1051941684, 1063219490],
        [1064109712, 1063006598, 1056211448, 1062307846, 1060510548,
         1058419146, 1033307040, 1061622336, 1053762360, 1039398624,
         1020728832, 1059299640],
    ],
    dtype=np.uint32,
).view(np.float32)
_SCORES_PADDED = np.full((8, 128), 2.0, np.float32)
_SCORES_PADDED[:B, :N_IN] = _SCORES


def _select_slots(read_score):
    src = [[jnp.int32(0)] * KEEP for _ in range(B)]
    for b in range(B):
        s = [read_score(b, i) for i in range(N_IN)]
        for j in range(N_IN):
            rank = jnp.int32(0)
            for k in range(N_IN):
                before = (s[k] < s[j]) | ((s[k] == s[j]) & (k < j))
                rank = rank + jnp.where(before, 1, 0)
            for p in range(KEEP):
                sel = rank == (KEEP + p)
                src[b][p] = jnp.where(sel, jnp.int32(j), src[b][p])
    return src


def _sc_body(in_hbm, scores_hbm, out_a, out_b, scores_v, buf, sem_in, sem_out):
    wid = lax.axis_index("s") * 2 + lax.axis_index("c")

    pltpu.sync_copy(scores_hbm, scores_v)
    s_vecs = [scores_v[b, pl.ds(0, 16)] for b in range(B)]
    src = _select_slots(lambda b, i: s_vecs[b][i])
    src_flat = [src[b][p] for b in range(B) for p in range(KEEP)]

    gathers = []
    meta = []
    for i in range(JOBS_PER_SUBCORE):
        g = wid + 32 * i
        j_idx = lax.div(g, N_CHUNKS)
        c_idx = lax.rem(g, N_CHUNKS)
        h0 = c_idx * HC
        b_idx = lax.div(j_idx, KEEP)
        s_row = jnp.int32(0)
        for j in range(B * KEEP):
            s_row = jnp.where(j_idx == j, src_flat[j], s_row)
        gathers.append(
            pltpu.async_copy(
                in_hbm.at[b_idx, s_row, pl.ds(h0, HC)], buf.at[i], sem_in
            )
        )
        meta.append((j_idx, b_idx, h0))
    stores = []
    for i in range(JOBS_PER_SUBCORE):
        j_idx, b_idx, h0 = meta[i]
        p_idx = lax.rem(j_idx, KEEP)
        gathers[i].wait()
        stores.append(
            pltpu.async_copy(
                buf.at[i], out_a.at[b_idx, p_idx, pl.ds(h0, HC)], sem_out
            )
        )
        stores.append(
            pltpu.async_copy(buf.at[i], out_b.at[j_idx, pl.ds(h0, HC)], sem_out)
        )
    for s_ in stores:
        s_.wait()


@jax.jit
def _sc_gather(xt, scores_tile):
    mesh = plsc.VectorSubcoreMesh(core_axis_name="c", subcore_axis_name="s")
    f = pl.kernel(
        _sc_body,
        out_type=(
            jax.ShapeDtypeStruct((B, KEEP, H, 32, 384), jnp.float32),
            jax.ShapeDtypeStruct((B * KEEP, H, 32, 384), jnp.float32),
        ),
        mesh=mesh,
        scratch_types=[
            pltpu.VMEM((8, 128), jnp.float32),
            pltpu.VMEM((JOBS_PER_SUBCORE, HC, 32, 384), jnp.float32),
            pltpu.SemaphoreType.DMA,
            pltpu.SemaphoreType.DMA,
        ],
        compiler_params=pltpu.CompilerParams(use_tc_tiling_on_sc=True),
    )
    return f(xt, scores_tile)


def kernel(image_latent):
    xt = jnp.transpose(image_latent, (0, 1, 3, 4, 2))
    ya, yb = _sc_gather(xt, jnp.asarray(_SCORES_PADDED))
    return (
        jnp.transpose(ya, (0, 1, 4, 2, 3)),
        jnp.transpose(yb, (0, 3, 1, 2)),
    )
